# baseline (speedup 1.0000x reference)
import functools

import jax
import jax.numpy as jnp
from jax import lax
from jax.experimental import pallas as pl
from jax.experimental.pallas import tpu as pltpu
from jax.experimental.pallas import tpu_sc as plsc

B, L, V, D = 4096, 200, 1000000, 64
H = 32
HP = 128

NC, NS, LANES = 2, 16, 16
NW = NC * NS
RPW = B // NW
LH1, LH2 = 128, 72

_mesh = plsc.VectorSubcoreMesh(core_axis_name="c", subcore_axis_name="s")


@functools.partial(
    pl.kernel,
    out_type=jax.ShapeDtypeStruct((B, H), jnp.float32),
    mesh=_mesh,
    compiler_params=pltpu.CompilerParams(use_tc_tiling_on_sc=True,
                                         needs_layout_passes=False),
    scratch_types=[
        pltpu.VMEM((RPW * L,), jnp.int32),
        pltpu.VMEM((RPW,), jnp.int32),
        pltpu.VMEM((2, L, HP), jnp.float32),
        pltpu.VMEM((RPW, H), jnp.float32),
        pltpu.SemaphoreType.DMA,
        pltpu.SemaphoreType.DMA,
    ],
)
def _bag_sum_kernel(batch_hbm, lengths_hbm, table_hbm, out_hbm,
                    idx_v, len_v, rows_v, sums_v, sem0, sem1):
    wid = lax.axis_index("s") * NC + lax.axis_index("c")
    base = wid * RPW

    pltpu.sync_copy(batch_hbm.at[pl.ds(base * L, RPW * L)], idx_v)
    pltpu.sync_copy(lengths_hbm.at[pl.ds(base, RPW)], len_v)

    def issue(r, buf, sem):
        off = pl.multiple_of(r * L, 8)
        pltpu.async_copy(table_hbm.at[idx_v.at[pl.ds(off, LH1)]],
                         rows_v.at[buf, pl.ds(0, LH1)], sem)
        off2 = pl.multiple_of(r * L + LH1, 8)
        pltpu.async_copy(table_hbm.at[idx_v.at[pl.ds(off2, LH2)]],
                         rows_v.at[buf, pl.ds(LH1, LH2)], sem)

    def wait(r, buf, sem):
        off = pl.multiple_of(r * L, 8)
        pltpu.make_async_copy(table_hbm.at[idx_v.at[pl.ds(off, LH1)]],
                              rows_v.at[buf, pl.ds(0, LH1)], sem).wait()
        off2 = pl.multiple_of(r * L + LH1, 8)
        pltpu.make_async_copy(table_hbm.at[idx_v.at[pl.ds(off2, LH2)]],
                              rows_v.at[buf, pl.ds(LH1, LH2)], sem).wait()

    def accumulate(r, buf):
        grp = pl.multiple_of((r // LANES) * LANES, LANES)
        lane = r - grp
        lv = len_v[pl.ds(grp, LANES)]
        lane_ids = lax.iota(jnp.int32, LANES)
        len_s = jnp.sum(jnp.where(lane_ids == lane, lv, 0))

        ngroups = (len_s + (LANES - 1)) // LANES

        def group_body(g, accs):
            gbase = g * LANES
            new = []
            for k in range(H // LANES):
                acc = accs[k]
                for j in range(LANES):
                    m = ((gbase + j) < len_s).astype(jnp.float32)
                    acc = acc + rows_v[buf, gbase + j, pl.ds(k * LANES, LANES)] * m
                new.append(acc)
            return tuple(new)

        zero = jnp.zeros((LANES,), jnp.float32)
        accs = lax.fori_loop(0, ngroups, group_body,
                             (zero,) * (H // LANES))
        for k in range(H // LANES):
            sums_v[r, pl.ds(k * LANES, LANES)] = accs[k]

    issue(0, 0, sem0)

    def row_pair(r2, _):
        r = r2 * 2
        issue(r + 1, 1, sem1)
        wait(r, 0, sem0)
        accumulate(r, 0)

        @pl.when(r + 2 < RPW)
        def _():
            issue(r + 2, 0, sem0)

        wait(r + 1, 1, sem1)
        accumulate(r + 1, 1)
        return 0

    lax.fori_loop(0, RPW // 2, row_pair, 0)
    pltpu.sync_copy(sums_v, out_hbm.at[pl.ds(base, RPW)])


def _mlp_body(sums_ref, inv_len_ref, b1_ref, W2_ref, b2_ref, out_ref):
    h = jnp.maximum(sums_ref[...] * inv_len_ref[...] + b1_ref[...], 0.0)
    out_ref[...] = (
        jnp.dot(h, W2_ref[...], preferred_element_type=jnp.float32)
        + b2_ref[...])


def kernel(batch, lengths, table, W1, b1, W2, b2):
    batch = batch.astype(jnp.int32).reshape(B * L)
    lengths = lengths.astype(jnp.int32)

    W1p = jnp.zeros((D, HP), jnp.float32).at[:, :H].set(W1)
    tw = jnp.dot(table, W1p, preferred_element_type=jnp.float32,
                 precision=lax.Precision.HIGHEST)

    sums = _bag_sum_kernel(batch, lengths, tw)

    inv_len = (1.0 / jnp.maximum(lengths, 1).astype(jnp.float32))[:, None]
    W2p = jnp.zeros((H, 128), jnp.float32).at[:, :2].set(W2)
    b2p = jnp.zeros((1, 128), jnp.float32).at[0, :2].set(b2)

    logits_p = pl.pallas_call(
        _mlp_body,
        out_shape=jax.ShapeDtypeStruct((B, 128), jnp.float32),
    )(sums, inv_len, b1[None, :], W2p, b2p)
    return logits_p[:, :2]


# speedup vs baseline: 2.2436x; 2.2436x over previous
import functools

import jax
import jax.numpy as jnp
from jax import lax
from jax.experimental import pallas as pl
from jax.experimental.pallas import tpu as pltpu
from jax.experimental.pallas import tpu_sc as plsc

B, L, V, D = 4096, 200, 1000000, 64
H = 32
HP = 128

NC, NS, LANES = 2, 16, 16
NW = NC * NS
RPW = B // NW
LH1, LH2 = 128, 72

_mesh = plsc.VectorSubcoreMesh(core_axis_name="c", subcore_axis_name="s")


@functools.partial(
    pl.kernel,
    out_type=jax.ShapeDtypeStruct((B, H), jnp.float32),
    mesh=_mesh,
    compiler_params=pltpu.CompilerParams(use_tc_tiling_on_sc=True,
                                         needs_layout_passes=False),
    scratch_types=[
        pltpu.VMEM((RPW * L,), jnp.int32),
        pltpu.VMEM((RPW,), jnp.int32),
        pltpu.VMEM((2, L, HP), jnp.float32),
        pltpu.VMEM((RPW, H), jnp.float32),
        pltpu.SemaphoreType.DMA,
        pltpu.SemaphoreType.DMA,
    ],
)
def _bag_sum_kernel(batch_hbm, lengths_hbm, table_hbm, out_hbm,
                    idx_v, len_v, rows_v, sums_v, sem0, sem1):
    wid = lax.axis_index("s") * NC + lax.axis_index("c")
    base = wid * RPW

    pltpu.sync_copy(batch_hbm.at[pl.ds(base * L, RPW * L)], idx_v)
    pltpu.sync_copy(lengths_hbm.at[pl.ds(base, RPW)], len_v)

    def issue(r, buf, sem):
        off = pl.multiple_of(r * L, 8)
        pltpu.async_copy(table_hbm.at[idx_v.at[pl.ds(off, LH1)]],
                         rows_v.at[buf, pl.ds(0, LH1)], sem)
        off2 = pl.multiple_of(r * L + LH1, 8)
        pltpu.async_copy(table_hbm.at[idx_v.at[pl.ds(off2, LH2)]],
                         rows_v.at[buf, pl.ds(LH1, LH2)], sem)

    def wait(r, buf, sem):
        off = pl.multiple_of(r * L, 8)
        pltpu.make_async_copy(table_hbm.at[idx_v.at[pl.ds(off, LH1)]],
                              rows_v.at[buf, pl.ds(0, LH1)], sem).wait()
        off2 = pl.multiple_of(r * L + LH1, 8)
        pltpu.make_async_copy(table_hbm.at[idx_v.at[pl.ds(off2, LH2)]],
                              rows_v.at[buf, pl.ds(LH1, LH2)], sem).wait()

    def accumulate(r, buf):
        grp = pl.multiple_of((r // LANES) * LANES, LANES)
        lane = r - grp
        lv = len_v[pl.ds(grp, LANES)]
        lane_ids = lax.iota(jnp.int32, LANES)
        len_s = jnp.sum(jnp.where(lane_ids == lane, lv, 0))

        ngroups = (len_s + (LANES - 1)) // LANES

        def group_body(g, accs):
            gbase = g * LANES
            new = []
            for k in range(H // LANES):
                acc = accs[k]
                for j in range(LANES):
                    m = ((gbase + j) < len_s).astype(jnp.float32)
                    acc = acc + rows_v[buf, gbase + j, pl.ds(k * LANES, LANES)] * m
                new.append(acc)
            return tuple(new)

        zero = jnp.zeros((LANES,), jnp.float32)
        accs = lax.fori_loop(0, ngroups, group_body,
                             (zero,) * (H // LANES))
        for k in range(H // LANES):
            sums_v[r, pl.ds(k * LANES, LANES)] = accs[k]

    issue(0, 0, sem0)

    def row_pair(r2, _):
        r = r2 * 2
        issue(r + 1, 1, sem1)
        wait(r, 0, sem0)
        accumulate(r, 0)

        @pl.when(r + 2 < RPW)
        def _():
            issue(r + 2, 0, sem0)

        wait(r + 1, 1, sem1)
        accumulate(r + 1, 1)
        return 0

    lax.fori_loop(0, RPW // 2, row_pair, 0)
    pltpu.sync_copy(sums_v, out_hbm.at[pl.ds(base, RPW)])


def _mlp_body(sums_ref, inv_len_ref, b1_ref, W2_ref, b2_ref, out_ref):
    h = jnp.maximum(sums_ref[...] * inv_len_ref[...] + b1_ref[...], 0.0)
    out_ref[...] = (
        jnp.dot(h, W2_ref[...], preferred_element_type=jnp.float32)
        + b2_ref[...])


def kernel(batch, lengths, table, W1, b1, W2, b2):
    batch = batch.astype(jnp.int32).reshape(B * L)
    lengths = lengths.astype(jnp.int32)

    W1p = jnp.zeros((D, HP), jnp.float32).at[:, :H].set(W1)
    tw = jnp.dot(table, W1p, preferred_element_type=jnp.float32)

    sums = _bag_sum_kernel(batch, lengths, tw)

    inv_len = (1.0 / jnp.maximum(lengths, 1).astype(jnp.float32))[:, None]
    W2p = jnp.zeros((H, 128), jnp.float32).at[:, :2].set(W2)
    b2p = jnp.zeros((1, 128), jnp.float32).at[0, :2].set(b2)

    logits_p = pl.pallas_call(
        _mlp_body,
        out_shape=jax.ShapeDtypeStruct((B, 128), jnp.float32),
    )(sums, inv_len, b1[None, :], W2p, b2p)
    return logits_p[:, :2]


# speedup vs baseline: 2.5628x; 1.1423x over previous
import functools

import jax
import jax.numpy as jnp
from jax import lax
from jax.experimental import pallas as pl
from jax.experimental.pallas import tpu as pltpu
from jax.experimental.pallas import tpu_sc as plsc

B, L, V, D = 4096, 200, 1000000, 64
H = 32
HP = 128

NC, NS, LANES = 2, 16, 16
NW = NC * NS
RPW = B // NW
LH1, LH2 = 128, 72

_mesh = plsc.VectorSubcoreMesh(core_axis_name="c", subcore_axis_name="s")


@functools.partial(
    pl.kernel,
    out_type=jax.ShapeDtypeStruct((B, H), jnp.float32),
    mesh=_mesh,
    compiler_params=pltpu.CompilerParams(use_tc_tiling_on_sc=True,
                                         needs_layout_passes=False),
    scratch_types=[
        pltpu.VMEM((RPW * L,), jnp.int32),
        pltpu.VMEM((RPW,), jnp.int32),
        pltpu.VMEM((2, L, HP), jnp.float32),
        pltpu.VMEM((RPW, H), jnp.float32),
        pltpu.SemaphoreType.DMA,
        pltpu.SemaphoreType.DMA,
    ],
)
def _bag_sum_kernel(batch_hbm, lengths_hbm, table_hbm, out_hbm,
                    idx_v, len_v, rows_v, sums_v, sem0, sem1):
    wid = lax.axis_index("s") * NC + lax.axis_index("c")
    base = wid * RPW

    pltpu.sync_copy(batch_hbm.at[pl.ds(base * L, RPW * L)], idx_v)
    pltpu.sync_copy(lengths_hbm.at[pl.ds(base, RPW)], len_v)

    def read_len(r):
        grp = pl.multiple_of((r // LANES) * LANES, LANES)
        lane = r - grp
        lv = len_v[pl.ds(grp, LANES)]
        lane_ids = lax.iota(jnp.int32, LANES)
        return jnp.sum(jnp.where(lane_ids == lane, lv, 0))

    BUCKETS = ((32, 0), (64, 0), (96, 0), (128, 0),
               (128, 32), (128, 64), (128, 72))

    def _for_bucket(r, buf, sem, len_s, fn):
        off = pl.multiple_of(r * L, 8)
        off2 = pl.multiple_of(r * L + LH1, 8)
        for i, (n1, n2) in enumerate(BUCKETS):
            lo = i * 32
            hi = (i + 1) * 32 if i < 6 else L

            @pl.when((len_s > lo) & (len_s <= hi))
            def _():
                fn(table_hbm.at[idx_v.at[pl.ds(off, n1)]],
                   rows_v.at[buf, pl.ds(0, n1)], sem)
                if n2:
                    fn(table_hbm.at[idx_v.at[pl.ds(off2, n2)]],
                       rows_v.at[buf, pl.ds(LH1, n2)], sem)

    def issue(r, buf, sem, len_s):
        _for_bucket(r, buf, sem, len_s, pltpu.async_copy)

    def wait(r, buf, sem, len_s):
        _for_bucket(r, buf, sem, len_s,
                    lambda s, d, m: pltpu.make_async_copy(s, d, m).wait())

    def accumulate(r, buf, len_s):
        ngroups = (len_s + (LANES - 1)) // LANES

        def group_body(g, accs):
            gbase = g * LANES
            new = []
            for k in range(H // LANES):
                acc = accs[k]
                for j in range(LANES):
                    m = ((gbase + j) < len_s).astype(jnp.float32)
                    acc = acc + rows_v[buf, gbase + j, pl.ds(k * LANES, LANES)] * m
                new.append(acc)
            return tuple(new)

        zero = jnp.zeros((LANES,), jnp.float32)
        accs = lax.fori_loop(0, ngroups, group_body,
                             (zero,) * (H // LANES))
        for k in range(H // LANES):
            sums_v[r, pl.ds(k * LANES, LANES)] = accs[k]

    len0 = read_len(0)
    issue(0, 0, sem0, len0)

    def row_pair(r2, carry):
        len_r = carry
        r = r2 * 2
        len_r1 = read_len(r + 1)
        issue(r + 1, 1, sem1, len_r1)
        wait(r, 0, sem0, len_r)
        accumulate(r, 0, len_r)

        len_r2 = read_len((r + 2) % RPW)

        @pl.when(r + 2 < RPW)
        def _():
            issue(r + 2, 0, sem0, len_r2)

        wait(r + 1, 1, sem1, len_r1)
        accumulate(r + 1, 1, len_r1)
        return len_r2

    lax.fori_loop(0, RPW // 2, row_pair, len0)
    pltpu.sync_copy(sums_v, out_hbm.at[pl.ds(base, RPW)])


def _mlp_body(sums_ref, inv_len_ref, b1_ref, W2_ref, b2_ref, out_ref):
    h = jnp.maximum(sums_ref[...] * inv_len_ref[...] + b1_ref[...], 0.0)
    out_ref[...] = (
        jnp.dot(h, W2_ref[...], preferred_element_type=jnp.float32)
        + b2_ref[...])


def kernel(batch, lengths, table, W1, b1, W2, b2):
    batch = batch.astype(jnp.int32).reshape(B * L)
    lengths = lengths.astype(jnp.int32)

    W1p = jnp.zeros((D, HP), jnp.float32).at[:, :H].set(W1)
    tw = jnp.dot(table, W1p, preferred_element_type=jnp.float32)

    sums = _bag_sum_kernel(batch, lengths, tw)

    inv_len = (1.0 / jnp.maximum(lengths, 1).astype(jnp.float32))[:, None]
    W2p = jnp.zeros((H, 128), jnp.float32).at[:, :2].set(W2)
    b2p = jnp.zeros((1, 128), jnp.float32).at[0, :2].set(b2)

    logits_p = pl.pallas_call(
        _mlp_body,
        out_shape=jax.ShapeDtypeStruct((B, 128), jnp.float32),
    )(sums, inv_len, b1[None, :], W2p, b2p)
    return logits_p[:, :2]


# speedup vs baseline: 3.0055x; 1.1728x over previous
import functools

import jax
import jax.numpy as jnp
from jax import lax
from jax.experimental import pallas as pl
from jax.experimental.pallas import tpu as pltpu
from jax.experimental.pallas import tpu_sc as plsc

B, L, V, D = 4096, 200, 1000000, 64
H = 32
HP = 128

NC, NS, LANES = 2, 16, 16
NW = NC * NS
RPW = B // NW
LH1, LH2 = 128, 72

_mesh = plsc.VectorSubcoreMesh(core_axis_name="c", subcore_axis_name="s")


@functools.partial(
    pl.kernel,
    out_type=jax.ShapeDtypeStruct((B, H), jnp.float32),
    mesh=_mesh,
    compiler_params=pltpu.CompilerParams(use_tc_tiling_on_sc=False,
                                         needs_layout_passes=False),
    scratch_types=[
        pltpu.VMEM((RPW * L,), jnp.int32),
        pltpu.VMEM((RPW,), jnp.int32),
        pltpu.VMEM((2, L, H), jnp.float32),
        pltpu.VMEM((RPW, H), jnp.float32),
        pltpu.SemaphoreType.DMA,
        pltpu.SemaphoreType.DMA,
    ],
)
def _bag_sum_kernel(batch_hbm, lengths_hbm, table_hbm, out_hbm,
                    idx_v, len_v, rows_v, sums_v, sem0, sem1):
    wid = lax.axis_index("s") * NC + lax.axis_index("c")
    base = wid * RPW

    pltpu.sync_copy(batch_hbm.at[pl.ds(base * L, RPW * L)], idx_v)
    pltpu.sync_copy(lengths_hbm.at[pl.ds(base, RPW)], len_v)

    def read_len(r):
        grp = pl.multiple_of((r // LANES) * LANES, LANES)
        lane = r - grp
        lv = len_v[pl.ds(grp, LANES)]
        lane_ids = lax.iota(jnp.int32, LANES)
        return jnp.sum(jnp.where(lane_ids == lane, lv, 0))

    BUCKETS = ((32, 0), (64, 0), (96, 0), (128, 0),
               (128, 32), (128, 64), (128, 72))

    def _for_bucket(r, buf, sem, len_s, fn):
        off = pl.multiple_of(r * L, 8)
        off2 = pl.multiple_of(r * L + LH1, 8)
        for i, (n1, n2) in enumerate(BUCKETS):
            lo = i * 32
            hi = (i + 1) * 32 if i < 6 else L

            @pl.when((len_s > lo) & (len_s <= hi))
            def _():
                fn(table_hbm.at[idx_v.at[pl.ds(off, n1)]],
                   rows_v.at[buf, pl.ds(0, n1)], sem)
                if n2:
                    fn(table_hbm.at[idx_v.at[pl.ds(off2, n2)]],
                       rows_v.at[buf, pl.ds(LH1, n2)], sem)

    def issue(r, buf, sem, len_s):
        _for_bucket(r, buf, sem, len_s, pltpu.async_copy)

    def wait(r, buf, sem, len_s):
        _for_bucket(r, buf, sem, len_s,
                    lambda s, d, m: pltpu.make_async_copy(s, d, m).wait())

    def accumulate(r, buf, len_s):
        ngroups = (len_s + (LANES - 1)) // LANES

        def group_body(g, accs):
            gbase = g * LANES
            new = []
            for k in range(H // LANES):
                acc = accs[k]
                for j in range(LANES):
                    m = ((gbase + j) < len_s).astype(jnp.float32)
                    acc = acc + rows_v[buf, gbase + j, pl.ds(k * LANES, LANES)] * m
                new.append(acc)
            return tuple(new)

        zero = jnp.zeros((LANES,), jnp.float32)
        accs = lax.fori_loop(0, ngroups, group_body,
                             (zero,) * (H // LANES))
        for k in range(H // LANES):
            sums_v[r, pl.ds(k * LANES, LANES)] = accs[k]

    len0 = read_len(0)
    issue(0, 0, sem0, len0)

    def row_pair(r2, carry):
        len_r = carry
        r = r2 * 2
        len_r1 = read_len(r + 1)
        issue(r + 1, 1, sem1, len_r1)
        wait(r, 0, sem0, len_r)
        accumulate(r, 0, len_r)

        len_r2 = read_len((r + 2) % RPW)

        @pl.when(r + 2 < RPW)
        def _():
            issue(r + 2, 0, sem0, len_r2)

        wait(r + 1, 1, sem1, len_r1)
        accumulate(r + 1, 1, len_r1)
        return len_r2

    lax.fori_loop(0, RPW // 2, row_pair, len0)
    pltpu.sync_copy(sums_v, out_hbm.at[pl.ds(base, RPW)])


def _mlp_body(sums_ref, inv_len_ref, b1_ref, W2_ref, b2_ref, out_ref):
    h = jnp.maximum(sums_ref[...] * inv_len_ref[...] + b1_ref[...], 0.0)
    out_ref[...] = (
        jnp.dot(h, W2_ref[...], preferred_element_type=jnp.float32)
        + b2_ref[...])


def kernel(batch, lengths, table, W1, b1, W2, b2):
    batch = batch.astype(jnp.int32).reshape(B * L) * 4
    lengths = lengths.astype(jnp.int32)

    W1p = jnp.tile(W1, (1, HP // H))
    tw = jnp.dot(table, W1p, preferred_element_type=jnp.float32)
    tw = tw.reshape(4 * V, H)

    sums = _bag_sum_kernel(batch, lengths, tw)

    inv_len = (1.0 / jnp.maximum(lengths, 1).astype(jnp.float32))[:, None]
    W2p = jnp.zeros((H, 128), jnp.float32).at[:, :2].set(W2)
    b2p = jnp.zeros((1, 128), jnp.float32).at[0, :2].set(b2)

    logits_p = pl.pallas_call(
        _mlp_body,
        out_shape=jax.ShapeDtypeStruct((B, 128), jnp.float32),
    )(sums, inv_len, b1[None, :], W2p, b2p)
    return logits_p[:, :2]
